# Initial kernel scaffold; baseline (speedup 1.0000x reference)
import jax
import jax.numpy as jnp
from jax.experimental import pallas as pl


def kernel(x, edge_index, W0, gru_w_ih, gru_w_hh, gru_b_ih, gru_b_hh, proj_w, proj_b, cls_w, cls_b):
    raise NotImplementedError("write your pallas kernel here")


# speedup vs baseline: 13.6244x; 13.6244x over previous
import functools

import jax
import jax.numpy as jnp
from jax import lax
from jax.experimental import pallas as pl
from jax.experimental.pallas import tpu as pltpu
from jax.experimental.pallas import tpu_sc as plsc

N = 10000
E = 320000
D = 128
NP = 10240
NC = 2
NS = 16
NW = NC * NS
CH = 128
CPW = 80
IG = 16
NGRP = CPW // IG
EPW = CH * CPW
EPAD = NW * EPW
RB = 512
GRID = NP // RB
ZPT = NP // NS


def _k1_body(x_ref, w0_ref, wih_ref, whh_ref, bih_ref, bhh_ref, out_ref, w_s):
    @pl.when(pl.program_id(0) == 0)
    def _():
        w0 = w0_ref[...]
        dn = (((1,), (1,)), ((), ()))
        gi = lax.dot_general(w0, wih_ref[...], dn,
                             preferred_element_type=jnp.float32) + bih_ref[...]
        gh = lax.dot_general(w0, whh_ref[...], dn,
                             preferred_element_type=jnp.float32) + bhh_ref[...]
        r = jax.nn.sigmoid(gi[:, :D] + gh[:, :D])
        z = jax.nn.sigmoid(gi[:, D:2 * D] + gh[:, D:2 * D])
        n = jnp.tanh(gi[:, 2 * D:] + r * gh[:, 2 * D:])
        w_s[...] = (1.0 - z) * n + z * w0

    out_ref[...] = jnp.dot(x_ref[...], w_s[...],
                           preferred_element_type=jnp.float32)


def _xw(x_pad, w0, wih, whh, bih, bhh):
    return pl.pallas_call(
        _k1_body,
        grid=(GRID,),
        in_specs=[
            pl.BlockSpec((RB, D), lambda i: (i, 0)),
            pl.BlockSpec((D, D), lambda i: (0, 0)),
            pl.BlockSpec((3 * D, D), lambda i: (0, 0)),
            pl.BlockSpec((3 * D, D), lambda i: (0, 0)),
            pl.BlockSpec((1, 3 * D), lambda i: (0, 0)),
            pl.BlockSpec((1, 3 * D), lambda i: (0, 0)),
        ],
        out_specs=pl.BlockSpec((RB, D), lambda i: (i, 0)),
        out_shape=jax.ShapeDtypeStruct((NP, D), jnp.float32),
        scratch_shapes=[pltpu.VMEM((D, D), jnp.float32)],
    )(x_pad, w0, wih, whh, bih, bhh)


def _deg_body(dst_hbm, out_hbm, dst_all, hist):
    c = lax.axis_index("c")
    s = lax.axis_index("s")
    wid = c * NS + s

    def z(i, carry):
        hist[pl.ds(i * 16, 16)] = jnp.zeros((16,), jnp.float32)
        return carry

    lax.fori_loop(0, NP // 16, z, 0)
    pltpu.sync_copy(dst_hbm.at[pl.ds(wid * EPW, EPW)], dst_all)

    def step(e, carry):
        idx = dst_all[pl.ds(e * 16, 16)]
        plsc.addupdate_scatter(hist, [idx],
                               jnp.full((16,), 1.0, jnp.float32))
        return carry

    lax.fori_loop(0, EPW // 16, step, 0)
    pltpu.sync_copy(hist, out_hbm.at[wid])


def _degrees(dst1d):
    mesh = plsc.VectorSubcoreMesh(core_axis_name="c", subcore_axis_name="s")
    f = pl.kernel(
        _deg_body,
        out_type=jax.ShapeDtypeStruct((NW, NP), jnp.float32),
        mesh=mesh,
        scratch_types=[
            pltpu.VMEM((EPW,), jnp.int32),
            pltpu.VMEM((NP,), jnp.float32),
        ],
        compiler_params=pltpu.CompilerParams(needs_layout_passes=False),
    )
    return f(dst1d)


def _dinv_col(dh_blk):
    ones32 = jnp.ones((NW, 1), jnp.float32)
    deg = lax.dot_general(dh_blk, ones32, (((0,), (0,)), ((), ())),
                          preferred_element_type=jnp.float32) + 1.0
    return lax.rsqrt(deg)


def _k3_body(xw_ref, dh_ref, y_ref):
    y_ref[...] = xw_ref[...] * _dinv_col(dh_ref[...])


def _scale(xw, deghist):
    return pl.pallas_call(
        _k3_body,
        grid=(GRID,),
        in_specs=[
            pl.BlockSpec((RB, D), lambda i: (i, 0)),
            pl.BlockSpec((NW, RB), lambda i: (0, i)),
        ],
        out_specs=pl.BlockSpec((RB, D), lambda i: (i, 0)),
        out_shape=jax.ShapeDtypeStruct((NP, D), jnp.float32),
    )(xw, deghist)


def _edge_pass(y, src2d, dst2d, zeros_rows):
    mesh = plsc.VectorSubcoreMesh(core_axis_name="c", subcore_axis_name="s")

    def body(y_hbm, src_hbm, dst_hbm, zero_hbm, out_hbm,
             src_v, dst0, dst1, rows0, rows1, acc_sh,
             semg0, semg1, semd0, semd1, sems0, sems1):
        c = lax.axis_index("c")
        s = lax.axis_index("s")
        wid = c * NS + s
        base = wid * CPW
        pltpu.sync_copy(zero_hbm.at[pl.ds(s * ZPT, ZPT)],
                        acc_sh.at[pl.ds(s * ZPT, ZPT)])
        pltpu.sync_copy(src_hbm.at[pl.ds(base, CPW)], src_v)
        plsc.subcore_barrier()

        pltpu.async_copy(dst_hbm.at[base], dst0, semd0)
        pltpu.async_copy(dst_hbm.at[base + 1], dst1, semd1)
        pltpu.async_copy(y_hbm.at[src_v.at[0]], rows0, semg0)
        pltpu.async_copy(y_hbm.at[src_v.at[1]], rows1, semg1)

        def step(i, carry):
            j0 = 2 * i
            j1 = j0 + 1
            more = i + 1 < CPW // 2
            pltpu.make_async_copy(y_hbm.at[src_v.at[j0]], rows0, semg0).wait()
            pltpu.make_async_copy(dst_hbm.at[base + j0], dst0, semd0).wait()
            pltpu.async_copy(rows0, acc_sh.at[dst0], sems0, add=True)

            pltpu.make_async_copy(y_hbm.at[src_v.at[j1]], rows1, semg1).wait()
            pltpu.make_async_copy(dst_hbm.at[base + j1], dst1, semd1).wait()
            pltpu.async_copy(rows1, acc_sh.at[dst1], sems1, add=True)

            pltpu.make_async_copy(rows0, acc_sh.at[dst0], sems0).wait()

            @pl.when(more)
            def _():
                pltpu.async_copy(dst_hbm.at[base + j0 + 2], dst0, semd0)
                pltpu.async_copy(y_hbm.at[src_v.at[j0 + 2]], rows0, semg0)

            pltpu.make_async_copy(rows1, acc_sh.at[dst1], sems1).wait()

            @pl.when(more)
            def _():
                pltpu.async_copy(dst_hbm.at[base + j1 + 2], dst1, semd1)
                pltpu.async_copy(y_hbm.at[src_v.at[j1 + 2]], rows1, semg1)

            return carry

        lax.fori_loop(0, CPW // 2, step, 0)
        plsc.subcore_barrier()
        pltpu.sync_copy(acc_sh.at[pl.ds(s * ZPT, ZPT)],
                        out_hbm.at[pl.ds(c * NP + s * ZPT, ZPT)])

    f = pl.kernel(
        body,
        out_type=jax.ShapeDtypeStruct((NC * NP, D), jnp.float32),
        mesh=mesh,
        scratch_types=[
            pltpu.VMEM((CPW, CH), jnp.int32),
            pltpu.VMEM((CH,), jnp.int32),
            pltpu.VMEM((CH,), jnp.int32),
            pltpu.VMEM((CH, D), jnp.float32),
            pltpu.VMEM((CH, D), jnp.float32),
            pltpu.VMEM_SHARED((NP, D), jnp.float32),
            pltpu.SemaphoreType.DMA,
            pltpu.SemaphoreType.DMA,
            pltpu.SemaphoreType.DMA,
            pltpu.SemaphoreType.DMA,
            pltpu.SemaphoreType.DMA,
            pltpu.SemaphoreType.DMA,
        ],
    )
    return f(y, src2d, dst2d, zeros_rows)


def _k5_body(a0_ref, a1_ref, y_ref, dh_ref, pw_ref, pb_ref, cw_ref,
             cb_ref, out_ref):
    h = (a0_ref[...] + a1_ref[...] + y_ref[...]) * _dinv_col(dh_ref[...])
    dn = (((1,), (1,)), ((), ()))
    t = lax.dot_general(h, pw_ref[...], dn,
                        preferred_element_type=jnp.float32) + pb_ref[...]
    t = jnp.maximum(t, 0.0)
    out_ref[...] = lax.dot_general(t, cw_ref[...], dn,
                                   preferred_element_type=jnp.float32) + cb_ref[...]


def _final(acc, y, deghist, proj_w, proj_b, cls_w_pad, cls_b_pad):
    return pl.pallas_call(
        _k5_body,
        grid=(GRID,),
        in_specs=[
            pl.BlockSpec((RB, D), lambda i: (i, 0)),
            pl.BlockSpec((RB, D), lambda i: (i + GRID, 0)),
            pl.BlockSpec((RB, D), lambda i: (i, 0)),
            pl.BlockSpec((NW, RB), lambda i: (0, i)),
            pl.BlockSpec((D, D), lambda i: (0, 0)),
            pl.BlockSpec((1, D), lambda i: (0, 0)),
            pl.BlockSpec((D, D), lambda i: (0, 0)),
            pl.BlockSpec((1, D), lambda i: (0, 0)),
        ],
        out_specs=pl.BlockSpec((RB, D), lambda i: (i, 0)),
        out_shape=jax.ShapeDtypeStruct((NP, D), jnp.float32),
    )(acc, acc, y, deghist, proj_w, proj_b, cls_w_pad, cls_b_pad)


def kernel(x, edge_index, W0, gru_w_ih, gru_w_hh, gru_b_ih, gru_b_hh,
           proj_w, proj_b, cls_w, cls_b):
    f32 = jnp.float32
    ei = edge_index.astype(jnp.int32)
    pad_e = EPAD - E
    src2d = jnp.concatenate(
        [ei[0], jnp.full((pad_e,), N, jnp.int32)]).reshape(EPAD // CH, CH)
    dst1d = jnp.concatenate([ei[1], jnp.full((pad_e,), N, jnp.int32)])
    dst2d = dst1d.reshape(EPAD // CH, CH)

    x_pad = jnp.pad(x, ((0, NP - N), (0, 0)))
    zeros_rows = jnp.zeros((NP, D), f32)
    bih = gru_b_ih.reshape(1, 3 * D)
    bhh = gru_b_hh.reshape(1, 3 * D)
    pb = proj_b.reshape(1, D)
    nc = cls_b.shape[0]
    cw_pad = jnp.zeros((D, D), f32).at[:nc].set(cls_w)
    cb_pad = jnp.zeros((1, D), f32).at[0, :nc].set(cls_b)

    xw = _xw(x_pad, W0, gru_w_ih, gru_w_hh, bih, bhh)
    deghist = _degrees(dst1d)
    y = _scale(xw, deghist)
    acc = _edge_pass(y, src2d, dst2d, zeros_rows)
    out = _final(acc, y, deghist, proj_w, pb, cw_pad, cb_pad)
    return out[:N, :nc]


# speedup vs baseline: 32.1158x; 2.3572x over previous
import functools

import jax
import jax.numpy as jnp
from jax import lax
from jax.experimental import pallas as pl
from jax.experimental.pallas import tpu as pltpu
from jax.experimental.pallas import tpu_sc as plsc

N = 10000
E = 320000
D = 128
NP = 10240
NC = 2
NS = 16
NW = NC * NS
CH = 128
CPW = 80
IG = 16
NGRP = CPW // IG
EPW = CH * CPW
EPAD = NW * EPW
RB = 512
GRID = NP // RB
ZPT = NP // NS


def _k1_body(x_ref, w0_ref, wih_ref, whh_ref, bih_ref, bhh_ref, out_ref, w_s):
    @pl.when(pl.program_id(0) == 0)
    def _():
        w0 = w0_ref[...]
        dn = (((1,), (1,)), ((), ()))
        gi = lax.dot_general(w0, wih_ref[...], dn,
                             preferred_element_type=jnp.float32) + bih_ref[...]
        gh = lax.dot_general(w0, whh_ref[...], dn,
                             preferred_element_type=jnp.float32) + bhh_ref[...]
        r = jax.nn.sigmoid(gi[:, :D] + gh[:, :D])
        z = jax.nn.sigmoid(gi[:, D:2 * D] + gh[:, D:2 * D])
        n = jnp.tanh(gi[:, 2 * D:] + r * gh[:, 2 * D:])
        w_s[...] = (1.0 - z) * n + z * w0

    out_ref[...] = jnp.dot(x_ref[...], w_s[...],
                           preferred_element_type=jnp.float32)


def _xw(x_pad, w0, wih, whh, bih, bhh):
    return pl.pallas_call(
        _k1_body,
        grid=(GRID,),
        in_specs=[
            pl.BlockSpec((RB, D), lambda i: (i, 0)),
            pl.BlockSpec((D, D), lambda i: (0, 0)),
            pl.BlockSpec((3 * D, D), lambda i: (0, 0)),
            pl.BlockSpec((3 * D, D), lambda i: (0, 0)),
            pl.BlockSpec((1, 3 * D), lambda i: (0, 0)),
            pl.BlockSpec((1, 3 * D), lambda i: (0, 0)),
        ],
        out_specs=pl.BlockSpec((RB, D), lambda i: (i, 0)),
        out_shape=jax.ShapeDtypeStruct((NP, D), jnp.float32),
        scratch_shapes=[pltpu.VMEM((D, D), jnp.float32)],
    )(x_pad, w0, wih, whh, bih, bhh)


def _deg_body(dst_hbm, out_hbm, dst_all, hist):
    c = lax.axis_index("c")
    s = lax.axis_index("s")
    wid = c * NS + s

    def z(i, carry):
        hist[pl.ds(i * 16, 16)] = jnp.zeros((16,), jnp.float32)
        return carry

    lax.fori_loop(0, NP // 16, z, 0)
    pltpu.sync_copy(dst_hbm.at[pl.ds(wid * EPW, EPW)], dst_all)

    def step(e, carry):
        idx = dst_all[pl.ds(e * 16, 16)]
        plsc.addupdate_scatter(hist, [idx],
                               jnp.full((16,), 1.0, jnp.float32))
        return carry

    lax.fori_loop(0, EPW // 16, step, 0)
    pltpu.sync_copy(hist, out_hbm.at[wid])


def _degrees(dst1d):
    mesh = plsc.VectorSubcoreMesh(core_axis_name="c", subcore_axis_name="s")
    f = pl.kernel(
        _deg_body,
        out_type=jax.ShapeDtypeStruct((NW, NP), jnp.float32),
        mesh=mesh,
        scratch_types=[
            pltpu.VMEM((EPW,), jnp.int32),
            pltpu.VMEM((NP,), jnp.float32),
        ],
        compiler_params=pltpu.CompilerParams(needs_layout_passes=False),
    )
    return f(dst1d)


def _dinv_col(dh_blk):
    ones32 = jnp.ones((NW, 1), jnp.float32)
    deg = lax.dot_general(dh_blk, ones32, (((0,), (0,)), ((), ())),
                          preferred_element_type=jnp.float32) + 1.0
    return lax.rsqrt(deg)


def _k3_body(xw_ref, dh_ref, y_ref):
    y_ref[...] = xw_ref[...] * _dinv_col(dh_ref[...])


def _scale(xw, deghist):
    return pl.pallas_call(
        _k3_body,
        grid=(GRID,),
        in_specs=[
            pl.BlockSpec((RB, D), lambda i: (i, 0)),
            pl.BlockSpec((NW, RB), lambda i: (0, i)),
        ],
        out_specs=pl.BlockSpec((RB, D), lambda i: (i, 0)),
        out_shape=jax.ShapeDtypeStruct((NP, D), jnp.float32),
    )(xw, deghist)


def _edge_pass(y, src2d, dst2d, zeros_rows):
    mesh = plsc.VectorSubcoreMesh(core_axis_name="c", subcore_axis_name="s")

    def body(y_hbm, src_hbm, dst_hbm, zero_hbm, out_hbm,
             src_v, dst0, dst1, rows0, rows1, acc_sh,
             semg0, semg1, semd0, semd1, sems0, sems1):
        c = lax.axis_index("c")
        s = lax.axis_index("s")
        wid = c * NS + s
        base = wid * CPW
        pltpu.sync_copy(zero_hbm.at[pl.ds(s * ZPT, ZPT)],
                        acc_sh.at[pl.ds(s * ZPT, ZPT)])
        pltpu.sync_copy(src_hbm.at[pl.ds(base, CPW)], src_v)
        plsc.subcore_barrier()

        pltpu.async_copy(dst_hbm.at[base], dst0, semd0)
        pltpu.async_copy(dst_hbm.at[base + 1], dst1, semd1)
        pltpu.async_copy(y_hbm.at[src_v.at[0]], rows0, semg0)
        pltpu.async_copy(y_hbm.at[src_v.at[1]], rows1, semg1)

        def step(i, carry):
            j0 = 2 * i
            j1 = j0 + 1
            more = i + 1 < CPW // 2
            pltpu.make_async_copy(y_hbm.at[src_v.at[j0]], rows0, semg0).wait()
            pltpu.make_async_copy(dst_hbm.at[base + j0], dst0, semd0).wait()
            pltpu.async_copy(rows0, acc_sh.at[dst0], sems0, add=True)

            pltpu.make_async_copy(y_hbm.at[src_v.at[j1]], rows1, semg1).wait()
            pltpu.make_async_copy(dst_hbm.at[base + j1], dst1, semd1).wait()
            pltpu.async_copy(rows1, acc_sh.at[dst1], sems1, add=True)

            pltpu.make_async_copy(rows0, acc_sh.at[dst0], sems0).wait()

            @pl.when(more)
            def _():
                pltpu.async_copy(dst_hbm.at[base + j0 + 2], dst0, semd0)
                pltpu.async_copy(y_hbm.at[src_v.at[j0 + 2]], rows0, semg0)

            pltpu.make_async_copy(rows1, acc_sh.at[dst1], sems1).wait()

            @pl.when(more)
            def _():
                pltpu.async_copy(dst_hbm.at[base + j1 + 2], dst1, semd1)
                pltpu.async_copy(y_hbm.at[src_v.at[j1 + 2]], rows1, semg1)

            return carry

        lax.fori_loop(0, CPW // 2, step, 0)
        plsc.subcore_barrier()
        pltpu.sync_copy(acc_sh.at[pl.ds(s * ZPT, ZPT)],
                        out_hbm.at[pl.ds(c * NP + s * ZPT, ZPT)])

    f = pl.kernel(
        body,
        out_type=jax.ShapeDtypeStruct((NC * NP, D), jnp.float32),
        mesh=mesh,
        scratch_types=[
            pltpu.VMEM((CPW, CH), jnp.int32),
            pltpu.VMEM((CH,), jnp.int32),
            pltpu.VMEM((CH,), jnp.int32),
            pltpu.VMEM((CH, D), jnp.float32),
            pltpu.VMEM((CH, D), jnp.float32),
            pltpu.VMEM_SHARED((NP, D), jnp.float32),
            pltpu.SemaphoreType.DMA,
            pltpu.SemaphoreType.DMA,
            pltpu.SemaphoreType.DMA,
            pltpu.SemaphoreType.DMA,
            pltpu.SemaphoreType.DMA,
            pltpu.SemaphoreType.DMA,
        ],
    )
    return f(y, src2d, dst2d, zeros_rows)


def _k5_body(a0_ref, a1_ref, y_ref, dh_ref, pw_ref, pb_ref, cw_ref,
             cb_ref, out_ref):
    h = (a0_ref[...] + a1_ref[...] + y_ref[...]) * _dinv_col(dh_ref[...])
    dn = (((1,), (1,)), ((), ()))
    t = lax.dot_general(h, pw_ref[...], dn,
                        preferred_element_type=jnp.float32) + pb_ref[...]
    t = jnp.maximum(t, 0.0)
    out_ref[...] = lax.dot_general(t, cw_ref[...], dn,
                                   preferred_element_type=jnp.float32) + cb_ref[...]


def _final(acc, y, deghist, proj_w, proj_b, cls_w_pad, cls_b_pad):
    return pl.pallas_call(
        _k5_body,
        grid=(GRID,),
        in_specs=[
            pl.BlockSpec((RB, D), lambda i: (i, 0)),
            pl.BlockSpec((RB, D), lambda i: (i + GRID, 0)),
            pl.BlockSpec((RB, D), lambda i: (i, 0)),
            pl.BlockSpec((NW, RB), lambda i: (0, i)),
            pl.BlockSpec((D, D), lambda i: (0, 0)),
            pl.BlockSpec((1, D), lambda i: (0, 0)),
            pl.BlockSpec((D, D), lambda i: (0, 0)),
            pl.BlockSpec((1, D), lambda i: (0, 0)),
        ],
        out_specs=pl.BlockSpec((RB, D), lambda i: (i, 0)),
        out_shape=jax.ShapeDtypeStruct((NP, D), jnp.float32),
    )(acc, acc, y, deghist, proj_w, proj_b, cls_w_pad, cls_b_pad)


def kernel(x, edge_index, W0, gru_w_ih, gru_w_hh, gru_b_ih, gru_b_hh,
           proj_w, proj_b, cls_w, cls_b):
    f32 = jnp.float32
    ei = edge_index.astype(jnp.int32)
    pad_e = EPAD - E
    pad_rows = N + (jnp.arange(pad_e, dtype=jnp.int32) % (NP - N))
    src2d = jnp.concatenate([ei[0], pad_rows]).reshape(EPAD // CH, CH)
    dst1d = jnp.concatenate([ei[1], pad_rows])
    dst2d = dst1d.reshape(EPAD // CH, CH)

    x_pad = jnp.pad(x, ((0, NP - N), (0, 0)))
    zeros_rows = jnp.zeros((NP, D), f32)
    bih = gru_b_ih.reshape(1, 3 * D)
    bhh = gru_b_hh.reshape(1, 3 * D)
    pb = proj_b.reshape(1, D)
    nc = cls_b.shape[0]
    cw_pad = jnp.zeros((D, D), f32).at[:nc].set(cls_w)
    cb_pad = jnp.zeros((1, D), f32).at[0, :nc].set(cls_b)

    xw = _xw(x_pad, W0, gru_w_ih, gru_w_hh, bih, bhh)
    deghist = _degrees(dst1d)
    y = _scale(xw, deghist)
    acc = _edge_pass(y, src2d, dst2d, zeros_rows)
    out = _final(acc, y, deghist, proj_w, pb, cw_pad, cb_pad)
    return out[:N, :nc]


# speedup vs baseline: 33.2902x; 1.0366x over previous
import functools

import jax
import jax.numpy as jnp
from jax import lax
from jax.experimental import pallas as pl
from jax.experimental.pallas import tpu as pltpu
from jax.experimental.pallas import tpu_sc as plsc

N = 10000
E = 320000
D = 128
NP = 10240
NC = 2
NS = 16
NW = NC * NS
CH = 128
CPW = 80
IG = 16
NGRP = CPW // IG
EPW = CH * CPW
EPAD = NW * EPW
RB = 512
GRID = NP // RB
ZPT = NP // NS


def _dinv_col(dh_blk):
    ones32 = jnp.ones((NW, 1), jnp.float32)
    deg = lax.dot_general(dh_blk, ones32, (((0,), (0,)), ((), ())),
                          preferred_element_type=jnp.float32) + 1.0
    return lax.rsqrt(deg)


def _k1_body(x_ref, dh_ref, w0_ref, wih_ref, whh_ref, bih_ref, bhh_ref,
             y_ref, w_s):
    @pl.when(pl.program_id(0) == 0)
    def _():
        w0 = w0_ref[...]
        dn = (((1,), (1,)), ((), ()))
        gi = lax.dot_general(w0, wih_ref[...], dn,
                             preferred_element_type=jnp.float32) + bih_ref[...]
        gh = lax.dot_general(w0, whh_ref[...], dn,
                             preferred_element_type=jnp.float32) + bhh_ref[...]
        r = jax.nn.sigmoid(gi[:, :D] + gh[:, :D])
        z = jax.nn.sigmoid(gi[:, D:2 * D] + gh[:, D:2 * D])
        n = jnp.tanh(gi[:, 2 * D:] + r * gh[:, 2 * D:])
        w_s[...] = (1.0 - z) * n + z * w0

    xw = jnp.dot(x_ref[...], w_s[...], preferred_element_type=jnp.float32)
    y_ref[...] = xw * _dinv_col(dh_ref[...])


def _scale(x_pad, deghist, w0, wih, whh, bih, bhh):
    return pl.pallas_call(
        _k1_body,
        grid=(GRID,),
        in_specs=[
            pl.BlockSpec((RB, D), lambda i: (i, 0)),
            pl.BlockSpec((NW, RB), lambda i: (0, i)),
            pl.BlockSpec((D, D), lambda i: (0, 0)),
            pl.BlockSpec((3 * D, D), lambda i: (0, 0)),
            pl.BlockSpec((3 * D, D), lambda i: (0, 0)),
            pl.BlockSpec((1, 3 * D), lambda i: (0, 0)),
            pl.BlockSpec((1, 3 * D), lambda i: (0, 0)),
        ],
        out_specs=pl.BlockSpec((RB, D), lambda i: (i, 0)),
        out_shape=jax.ShapeDtypeStruct((NP, D), jnp.float32),
        scratch_shapes=[pltpu.VMEM((D, D), jnp.float32)],
    )(x_pad, deghist, w0, wih, whh, bih, bhh)


def _deg_body(dst_hbm, out_hbm, dst_all, hist):
    c = lax.axis_index("c")
    s = lax.axis_index("s")
    wid = c * NS + s

    def z(i, carry):
        hist[pl.ds(i * 16, 16)] = jnp.zeros((16,), jnp.float32)
        return carry

    lax.fori_loop(0, NP // 16, z, 0)
    pltpu.sync_copy(dst_hbm.at[pl.ds(wid * EPW, EPW)], dst_all)

    def step(e, carry):
        idx = dst_all[pl.ds(e * 16, 16)]
        plsc.addupdate_scatter(hist, [idx],
                               jnp.full((16,), 1.0, jnp.float32))
        return carry

    lax.fori_loop(0, EPW // 16, step, 0)
    pltpu.sync_copy(hist, out_hbm.at[wid])


def _degrees(dst1d):
    mesh = plsc.VectorSubcoreMesh(core_axis_name="c", subcore_axis_name="s")
    f = pl.kernel(
        _deg_body,
        out_type=jax.ShapeDtypeStruct((NW, NP), jnp.float32),
        mesh=mesh,
        scratch_types=[
            pltpu.VMEM((EPW,), jnp.int32),
            pltpu.VMEM((NP,), jnp.float32),
        ],
        compiler_params=pltpu.CompilerParams(needs_layout_passes=False),
    )
    return f(dst1d)


def _edge_pass(y, src2d, dst2d, zeros_rows):
    mesh = plsc.VectorSubcoreMesh(core_axis_name="c", subcore_axis_name="s")

    def body(y_hbm, src_hbm, dst_hbm, zero_hbm, out_hbm,
             src_v, dst0, dst1, rows0, rows1, acc_sh,
             semg0, semg1, semd0, semd1, sems0, sems1):
        c = lax.axis_index("c")
        s = lax.axis_index("s")
        wid = c * NS + s
        base = wid * CPW
        pltpu.sync_copy(zero_hbm.at[pl.ds(s * ZPT, ZPT)],
                        acc_sh.at[pl.ds(s * ZPT, ZPT)])
        pltpu.sync_copy(src_hbm.at[pl.ds(base, CPW)], src_v)
        plsc.subcore_barrier()

        pltpu.async_copy(dst_hbm.at[base], dst0, semd0)
        pltpu.async_copy(dst_hbm.at[base + 1], dst1, semd1)
        pltpu.async_copy(y_hbm.at[src_v.at[0]], rows0, semg0)
        pltpu.async_copy(y_hbm.at[src_v.at[1]], rows1, semg1)

        def step(i, carry):
            j0 = 2 * i
            j1 = j0 + 1
            more = i + 1 < CPW // 2
            pltpu.make_async_copy(y_hbm.at[src_v.at[j0]], rows0, semg0).wait()
            pltpu.make_async_copy(dst_hbm.at[base + j0], dst0, semd0).wait()
            pltpu.async_copy(rows0, acc_sh.at[dst0], sems0, add=True)

            pltpu.make_async_copy(y_hbm.at[src_v.at[j1]], rows1, semg1).wait()
            pltpu.make_async_copy(dst_hbm.at[base + j1], dst1, semd1).wait()
            pltpu.async_copy(rows1, acc_sh.at[dst1], sems1, add=True)

            pltpu.make_async_copy(rows0, acc_sh.at[dst0], sems0).wait()

            @pl.when(more)
            def _():
                pltpu.async_copy(dst_hbm.at[base + j0 + 2], dst0, semd0)
                pltpu.async_copy(y_hbm.at[src_v.at[j0 + 2]], rows0, semg0)

            pltpu.make_async_copy(rows1, acc_sh.at[dst1], sems1).wait()

            @pl.when(more)
            def _():
                pltpu.async_copy(dst_hbm.at[base + j1 + 2], dst1, semd1)
                pltpu.async_copy(y_hbm.at[src_v.at[j1 + 2]], rows1, semg1)

            return carry

        lax.fori_loop(0, CPW // 2, step, 0)
        plsc.subcore_barrier()
        pltpu.sync_copy(acc_sh.at[pl.ds(s * ZPT, ZPT)],
                        out_hbm.at[pl.ds(c * NP + s * ZPT, ZPT)])

    f = pl.kernel(
        body,
        out_type=jax.ShapeDtypeStruct((NC * NP, D), jnp.float32),
        mesh=mesh,
        scratch_types=[
            pltpu.VMEM((CPW, CH), jnp.int32),
            pltpu.VMEM((CH,), jnp.int32),
            pltpu.VMEM((CH,), jnp.int32),
            pltpu.VMEM((CH, D), jnp.float32),
            pltpu.VMEM((CH, D), jnp.float32),
            pltpu.VMEM_SHARED((NP, D), jnp.float32),
            pltpu.SemaphoreType.DMA,
            pltpu.SemaphoreType.DMA,
            pltpu.SemaphoreType.DMA,
            pltpu.SemaphoreType.DMA,
            pltpu.SemaphoreType.DMA,
            pltpu.SemaphoreType.DMA,
        ],
    )
    return f(y, src2d, dst2d, zeros_rows)


def _k5_body(a0_ref, a1_ref, y_ref, dh_ref, pw_ref, pb_ref, cw_ref,
             cb_ref, out_ref):
    h = (a0_ref[...] + a1_ref[...] + y_ref[...]) * _dinv_col(dh_ref[...])
    dn = (((1,), (1,)), ((), ()))
    t = lax.dot_general(h, pw_ref[...], dn,
                        preferred_element_type=jnp.float32) + pb_ref[...]
    t = jnp.maximum(t, 0.0)
    out_ref[...] = lax.dot_general(t, cw_ref[...], dn,
                                   preferred_element_type=jnp.float32) + cb_ref[...]


def _final(acc, y, deghist, proj_w, proj_b, cls_w_pad, cls_b_pad):
    return pl.pallas_call(
        _k5_body,
        grid=(GRID,),
        in_specs=[
            pl.BlockSpec((RB, D), lambda i: (i, 0)),
            pl.BlockSpec((RB, D), lambda i: (i + GRID, 0)),
            pl.BlockSpec((RB, D), lambda i: (i, 0)),
            pl.BlockSpec((NW, RB), lambda i: (0, i)),
            pl.BlockSpec((D, D), lambda i: (0, 0)),
            pl.BlockSpec((1, D), lambda i: (0, 0)),
            pl.BlockSpec((D, D), lambda i: (0, 0)),
            pl.BlockSpec((1, D), lambda i: (0, 0)),
        ],
        out_specs=pl.BlockSpec((RB, D), lambda i: (i, 0)),
        out_shape=jax.ShapeDtypeStruct((NP, D), jnp.float32),
    )(acc, acc, y, deghist, proj_w, proj_b, cls_w_pad, cls_b_pad)


def kernel(x, edge_index, W0, gru_w_ih, gru_w_hh, gru_b_ih, gru_b_hh,
           proj_w, proj_b, cls_w, cls_b):
    f32 = jnp.float32
    ei = edge_index.astype(jnp.int32)
    pad_e = EPAD - E
    pad_rows = N + (jnp.arange(pad_e, dtype=jnp.int32) % (NP - N))
    src2d = jnp.concatenate([ei[0], pad_rows]).reshape(EPAD // CH, CH)
    dst1d = jnp.concatenate([ei[1], pad_rows])
    dst2d = dst1d.reshape(EPAD // CH, CH)

    x_pad = jnp.pad(x, ((0, NP - N), (0, 0)))
    zeros_rows = jnp.zeros((NP, D), f32)
    bih = gru_b_ih.reshape(1, 3 * D)
    bhh = gru_b_hh.reshape(1, 3 * D)
    pb = proj_b.reshape(1, D)
    nc = cls_b.shape[0]
    cw_pad = jnp.zeros((D, D), f32).at[:nc].set(cls_w)
    cb_pad = jnp.zeros((1, D), f32).at[0, :nc].set(cls_b)

    deghist = _degrees(dst1d)
    y = _scale(x_pad, deghist, W0, gru_w_ih, gru_w_hh, bih, bhh)
    acc = _edge_pass(y, src2d, dst2d, zeros_rows)
    out = _final(acc, y, deghist, proj_w, pb, cw_pad, cb_pad)
    return out[:N, :nc]
